# Initial kernel scaffold; baseline (speedup 1.0000x reference)
import jax
import jax.numpy as jnp
from jax.experimental import pallas as pl


def kernel(key_states, value_states, layer_idx):
    raise NotImplementedError("write your pallas kernel here")


# speedup vs baseline: 8.3997x; 8.3997x over previous
import functools

import numpy as np
import jax
import jax.numpy as jnp
from jax import lax
from jax.experimental import pallas as pl
from jax.experimental.pallas import tpu as pltpu
from jax.experimental.pallas import tpu_sc as plsc

_S = 8192
_W = 512
_NSINK = 4
_NCAS = _S // _W

_CHUNK = 128
_ZROWS = 256


def _cascade_layout(T):
    cache = [-1] * _S
    start = [0] * _NCAS
    stored = [0] * _NCAS
    do_every = [2 ** i for i in range(_NCAS)]
    sink_ids = []
    seen = 0
    for t in range(T):
        seen += 1
        if len(sink_ids) < _NSINK:
            sink_ids.append(t)
            continue
        do_cache = [(seen - 1 - _NSINK) % do_every[i] == 0 for i in range(_NCAS)]
        tok = t
        ci = 0
        while tok is not None and ci < _NCAS:
            l = _W * ci
            if do_cache[ci]:
                if stored[ci] < _W:
                    cache[l + (start[ci] + stored[ci]) % _W] = tok
                    stored[ci] += 1
                    tok = None
                else:
                    s = l + start[ci]
                    evicted = cache[s]
                    cache[s] = tok
                    start[ci] = (start[ci] + 1) % _W
                    tok = evicted
                    ci += 1
            else:
                if stored[ci] > 0:
                    s = l + (start[ci] + stored[ci] - 1) % _W
                    cache[s] = tok
                tok = None
    slots = [i for i, v in enumerate(cache) if v >= 0]
    toks = [cache[i] for i in slots]
    return (np.asarray(sink_ids, np.int64), np.asarray(slots, np.int64),
            np.asarray(toks, np.int64))


@functools.lru_cache(maxsize=None)
def _gather_plan(T, H):
    sink_ids, slots, toks = _cascade_layout(T)
    dst = np.concatenate([np.arange(_NSINK), _NSINK + slots])
    src = np.concatenate([sink_ids, toks])
    order = np.argsort(dst, kind="stable")
    dst, src = dst[order], src[order]
    n = len(dst)
    assert np.array_equal(dst, np.arange(n)), "occupied slots not contiguous"

    reg = _NSINK + _S
    fpad = reg % 8
    bpad = (-n) % 8
    assert (n + bpad) % 8 == 0 and (fpad + n) % 8 == 0

    ent_k = list(src) + [None] * bpad
    ent_v = [None] * fpad + list(src)
    assert len(ent_k) == len(ent_v)
    ne = len(ent_k)
    nchunk = -(-ne // _CHUNK)
    last_m = ne - (nchunk - 1) * _CHUNK
    pad_total = nchunk * _CHUNK - ne

    origin = (0, reg - fpad)
    cover_end = (ne, reg - fpad + ne)

    idx = np.zeros((2 * H, nchunk, _CHUNK), np.int32)
    vzero = ([], [])
    for part, ents in enumerate((ent_k, ent_v)):
        full = ents + [None] * pad_total
        for j in range(nchunk):
            for r in range(_CHUNK):
                e = full[j * _CHUNK + r]
                if e is None and j * _CHUNK + r < ne:
                    vzero[part].append((j, r))
        base_idx = np.asarray([0 if e is None else e for e in full], np.int64)
        for h in range(H):
            idx[part * H + h] = (base_idx + h * T).astype(np.int32).reshape(
                nchunk, _CHUNK)

    zruns = ([], [])
    zend = (reg - fpad, 2 * reg)
    for part in range(2):
        z = cover_end[part]
        while z < zend[part]:
            m = min(_ZROWS, zend[part] - z)
            zruns[part].append((z, m))
            z += m
        assert cover_end[part] % 8 == 0

    return dict(idx=idx, nchunk=nchunk, last_m=last_m, origin=origin,
                vzero=vzero, zruns=zruns)


@functools.lru_cache(maxsize=None)
def _build_kernel(T, H, D):
    plan = _gather_plan(T, H)
    reg = _NSINK + _S
    outt = 2 * reg
    nunits = 2 * H
    assert nunits == 32, "one unit per TEC vector subcore"
    nchunk = plan["nchunk"]
    last_m = plan["last_m"]

    mesh = plsc.VectorSubcoreMesh(core_axis_name="c", subcore_axis_name="s")

    @functools.partial(
        pl.kernel,
        out_type=jax.ShapeDtypeStruct((H * outt, D), jnp.float32),
        mesh=mesh,
        scratch_types=[
            pltpu.VMEM((nchunk, _CHUNK), jnp.int32),
            pltpu.VMEM((_CHUNK, D), jnp.float32),
            pltpu.VMEM((_CHUNK, D), jnp.float32),
            pltpu.VMEM((_ZROWS, D), jnp.float32),
            pltpu.SemaphoreType.DMA,
            pltpu.SemaphoreType.DMA,
            pltpu.SemaphoreType.DMA,
            pltpu.SemaphoreType.DMA,
            pltpu.SemaphoreType.DMA,
        ],
    )
    def cache_fill(key_hbm, val_hbm, idx_hbm, zeros_hbm, out_hbm,
                   idx_v, rows0, rows1, zbuf, gsem0, gsem1, ssem0, ssem1, zsem):
        c = lax.axis_index("c")
        s = lax.axis_index("s")
        w = s * 2 + c
        part = w // H
        head = w % H
        hbase = head * outt

        pltpu.sync_copy(idx_hbm.at[w], idx_v)
        pltpu.sync_copy(zeros_hbm, zbuf)

        rows = (rows0, rows1)
        gsems = (gsem0, gsem1)
        ssems = (ssem0, ssem1)
        zero16 = jnp.zeros((16,), jnp.float32)

        def run(table, part_i):
            origin = plan["origin"][part_i]
            vzero = plan["vzero"][part_i]
            zdescs = [
                pltpu.async_copy(zbuf.at[pl.ds(0, m)],
                                 out_hbm.at[pl.ds(hbase + z, m)], zsem)
                for (z, m) in plan["zruns"][part_i]
            ]
            pending = [None, None]
            for j in range(nchunk):
                b = j & 1
                if pending[b] is not None:
                    pending[b].wait()
                pltpu.async_copy(table.at[idx_v.at[j]], rows[b],
                                 gsems[b]).wait()
                for (jj, r) in vzero:
                    if jj == j:
                        for k in range(D // 16):
                            rows[b][r, pl.ds(k * 16, 16)] = zero16
                m = _CHUNK if j < nchunk - 1 else last_m
                pending[b] = pltpu.async_copy(
                    rows[b].at[pl.ds(0, m)],
                    out_hbm.at[pl.ds(hbase + origin + j * _CHUNK, m)],
                    ssems[b])
            for b in (0, 1):
                if pending[b] is not None:
                    pending[b].wait()
            for d in zdescs:
                d.wait()

        @pl.when(part == 0)
        def _():
            run(key_hbm, 0)

        @pl.when(part == 1)
        def _():
            run(val_hbm, 1)

    return cache_fill


def kernel(key_states, value_states, layer_idx):
    del layer_idx
    B, H, T, D = key_states.shape
    assert B == 1
    plan = _gather_plan(T, H)
    fn = _build_kernel(T, H, D)
    out_flat = fn(
        key_states.reshape(H * T, D),
        value_states.reshape(H * T, D),
        jnp.asarray(plan["idx"]),
        jnp.zeros((_ZROWS, D), jnp.float32),
    )
    outt = 2 * (_NSINK + _S)
    return out_flat.reshape(B, H, outt, D)


# speedup vs baseline: 8.6658x; 1.0317x over previous
import functools

import numpy as np
import jax
import jax.numpy as jnp
from jax import lax
from jax.experimental import pallas as pl
from jax.experimental.pallas import tpu as pltpu
from jax.experimental.pallas import tpu_sc as plsc

_S = 8192
_W = 512
_NSINK = 4
_NCAS = _S // _W

_CHUNK = 128
_ZROWS = 448
_NBUF = 4


def _cascade_layout(T):
    cache = [-1] * _S
    start = [0] * _NCAS
    stored = [0] * _NCAS
    do_every = [2 ** i for i in range(_NCAS)]
    sink_ids = []
    seen = 0
    for t in range(T):
        seen += 1
        if len(sink_ids) < _NSINK:
            sink_ids.append(t)
            continue
        do_cache = [(seen - 1 - _NSINK) % do_every[i] == 0 for i in range(_NCAS)]
        tok = t
        ci = 0
        while tok is not None and ci < _NCAS:
            l = _W * ci
            if do_cache[ci]:
                if stored[ci] < _W:
                    cache[l + (start[ci] + stored[ci]) % _W] = tok
                    stored[ci] += 1
                    tok = None
                else:
                    s = l + start[ci]
                    evicted = cache[s]
                    cache[s] = tok
                    start[ci] = (start[ci] + 1) % _W
                    tok = evicted
                    ci += 1
            else:
                if stored[ci] > 0:
                    s = l + (start[ci] + stored[ci] - 1) % _W
                    cache[s] = tok
                tok = None
    slots = [i for i, v in enumerate(cache) if v >= 0]
    toks = [cache[i] for i in slots]
    return (np.asarray(sink_ids, np.int64), np.asarray(slots, np.int64),
            np.asarray(toks, np.int64))


@functools.lru_cache(maxsize=None)
def _gather_plan(T, H):
    sink_ids, slots, toks = _cascade_layout(T)
    dst = np.concatenate([np.arange(_NSINK), _NSINK + slots])
    src = np.concatenate([sink_ids, toks])
    order = np.argsort(dst, kind="stable")
    dst, src = dst[order], src[order]
    n = len(dst)
    assert np.array_equal(dst, np.arange(n)), "occupied slots not contiguous"

    reg = _NSINK + _S
    fpad = reg % 8
    bpad = (-n) % 8
    assert (n + bpad) % 8 == 0 and (fpad + n) % 8 == 0

    ent_k = list(src) + [None] * bpad
    ent_v = [None] * fpad + list(src)
    assert len(ent_k) == len(ent_v)
    ne = len(ent_k)
    nchunk = -(-ne // _CHUNK)
    last_m = ne - (nchunk - 1) * _CHUNK
    pad_total = nchunk * _CHUNK - ne

    origin = (0, reg - fpad)
    cover_end = (ne, reg - fpad + ne)

    idx = np.zeros((2 * H, nchunk, _CHUNK), np.int32)
    vzero = ([], [])
    for part, ents in enumerate((ent_k, ent_v)):
        full = ents + [None] * pad_total
        for j in range(nchunk):
            for r in range(_CHUNK):
                e = full[j * _CHUNK + r]
                if e is None and j * _CHUNK + r < ne:
                    vzero[part].append((j, r))
        base_idx = np.asarray([0 if e is None else e for e in full], np.int64)
        for h in range(H):
            idx[part * H + h] = (base_idx + h * T).astype(np.int32).reshape(
                nchunk, _CHUNK)

    zruns = ([], [])
    zend = (reg - fpad, 2 * reg)
    for part in range(2):
        z = cover_end[part]
        while z < zend[part]:
            m = min(_ZROWS, zend[part] - z)
            zruns[part].append((z, m))
            z += m
        assert cover_end[part] % 8 == 0

    return dict(idx=idx, nchunk=nchunk, last_m=last_m, origin=origin,
                vzero=vzero, zruns=zruns)


@functools.lru_cache(maxsize=None)
def _build_kernel(T, H, D):
    plan = _gather_plan(T, H)
    reg = _NSINK + _S
    outt = 2 * reg
    nunits = 2 * H
    assert nunits == 32, "one unit per TEC vector subcore"
    nchunk = plan["nchunk"]
    last_m = plan["last_m"]

    mesh = plsc.VectorSubcoreMesh(core_axis_name="c", subcore_axis_name="s")

    @functools.partial(
        pl.kernel,
        out_type=jax.ShapeDtypeStruct((H * outt, D), jnp.float32),
        mesh=mesh,
        scratch_types=(
            [pltpu.VMEM((nchunk, _CHUNK), jnp.int32)]
            + [pltpu.VMEM((_CHUNK, D), jnp.float32)] * _NBUF
            + [pltpu.VMEM((_ZROWS, D), jnp.float32)]
            + [pltpu.SemaphoreType.DMA] * (2 * _NBUF + 1)
        ),
    )
    def cache_fill(key_hbm, val_hbm, idx_hbm, zeros_hbm, out_hbm,
                   idx_v, *scratch):
        rows = scratch[:_NBUF]
        zbuf = scratch[_NBUF]
        gsems = scratch[_NBUF + 1:2 * _NBUF + 1]
        ssems = scratch[2 * _NBUF + 1:3 * _NBUF + 1]
        zsem = scratch[3 * _NBUF + 1]
        c = lax.axis_index("c")
        s = lax.axis_index("s")
        w = s * 2 + c
        part = w // H
        head = w % H
        hbase = head * outt

        pltpu.sync_copy(zeros_hbm, zbuf)
        pltpu.sync_copy(idx_hbm.at[w], idx_v)

        zero16 = jnp.zeros((16,), jnp.float32)

        def run(table, part_i):
            origin = plan["origin"][part_i]
            vzero = plan["vzero"][part_i]
            zdescs = [
                pltpu.async_copy(zbuf.at[pl.ds(0, m)],
                                 out_hbm.at[pl.ds(hbase + z, m)], zsem)
                for (z, m) in plan["zruns"][part_i]
            ]

            def fire_gather(j):
                return pltpu.async_copy(table.at[idx_v.at[j]],
                                        rows[j % _NBUF], gsems[j % _NBUF])

            def fire_store(j):
                m = _CHUNK if j < nchunk - 1 else last_m
                return pltpu.async_copy(
                    rows[j % _NBUF].at[pl.ds(0, m)],
                    out_hbm.at[pl.ds(hbase + origin + j * _CHUNK, m)],
                    ssems[j % _NBUF])

            gd = [None] * nchunk
            sd = [None] * nchunk
            for j in range(min(_NBUF - 1, nchunk)):
                gd[j] = fire_gather(j)
            for j in range(nchunk):
                gd[j].wait()
                for (jj, r) in vzero:
                    if jj == j:
                        for k in range(D // 16):
                            rows[j % _NBUF][r, pl.ds(k * 16, 16)] = zero16
                sd[j] = fire_store(j)
                nxt = j + _NBUF - 1
                if nxt < nchunk and gd[nxt] is None:
                    prev = nxt - _NBUF
                    if prev >= 0:
                        sd[prev].wait()
                        sd[prev] = None
                    gd[nxt] = fire_gather(nxt)
            for j in range(nchunk):
                if sd[j] is not None:
                    sd[j].wait()
            for d in zdescs:
                d.wait()

        @pl.when(part == 0)
        def _():
            run(key_hbm, 0)

        @pl.when(part == 1)
        def _():
            run(val_hbm, 1)

    return cache_fill


def kernel(key_states, value_states, layer_idx):
    del layer_idx
    B, H, T, D = key_states.shape
    assert B == 1
    plan = _gather_plan(T, H)
    fn = _build_kernel(T, H, D)
    out_flat = fn(
        key_states.reshape(H * T, D),
        value_states.reshape(H * T, D),
        jnp.asarray(plan["idx"]),
        jnp.zeros((_ZROWS, D), jnp.float32),
    )
    outt = 2 * (_NSINK + _S)
    return out_flat.reshape(B, H, outt, D)


# speedup vs baseline: 8.9375x; 1.0314x over previous
import functools

import numpy as np
import jax
import jax.numpy as jnp
from jax import lax
from jax.experimental import pallas as pl
from jax.experimental.pallas import tpu as pltpu
from jax.experimental.pallas import tpu_sc as plsc

_S = 8192
_W = 512
_NSINK = 4
_NCAS = _S // _W

_CHUNK = 128
_ZROWS = 512
_NBUF = 3


def _cascade_layout(T):
    cache = [-1] * _S
    start = [0] * _NCAS
    stored = [0] * _NCAS
    do_every = [2 ** i for i in range(_NCAS)]
    sink_ids = []
    seen = 0
    for t in range(T):
        seen += 1
        if len(sink_ids) < _NSINK:
            sink_ids.append(t)
            continue
        do_cache = [(seen - 1 - _NSINK) % do_every[i] == 0 for i in range(_NCAS)]
        tok = t
        ci = 0
        while tok is not None and ci < _NCAS:
            l = _W * ci
            if do_cache[ci]:
                if stored[ci] < _W:
                    cache[l + (start[ci] + stored[ci]) % _W] = tok
                    stored[ci] += 1
                    tok = None
                else:
                    s = l + start[ci]
                    evicted = cache[s]
                    cache[s] = tok
                    start[ci] = (start[ci] + 1) % _W
                    tok = evicted
                    ci += 1
            else:
                if stored[ci] > 0:
                    s = l + (start[ci] + stored[ci] - 1) % _W
                    cache[s] = tok
                tok = None
    slots = [i for i, v in enumerate(cache) if v >= 0]
    toks = [cache[i] for i in slots]
    return (np.asarray(sink_ids, np.int64), np.asarray(slots, np.int64),
            np.asarray(toks, np.int64))


@functools.lru_cache(maxsize=None)
def _gather_plan(T, H):
    sink_ids, slots, toks = _cascade_layout(T)
    dst = np.concatenate([np.arange(_NSINK), _NSINK + slots])
    src = np.concatenate([sink_ids, toks])
    order = np.argsort(dst, kind="stable")
    dst, src = dst[order], src[order]
    n = len(dst)
    assert np.array_equal(dst, np.arange(n)), "occupied slots not contiguous"

    reg = _NSINK + _S
    fpad = reg % 8
    bpad = (-n) % 8
    assert (n + bpad) % 8 == 0 and (fpad + n) % 8 == 0

    ent_k = list(src) + [None] * bpad
    ent_v = [None] * fpad + list(src)
    assert len(ent_k) == len(ent_v)
    ne = len(ent_k)
    nchunk = -(-ne // _CHUNK)
    last_m = ne - (nchunk - 1) * _CHUNK
    pad_total = nchunk * _CHUNK - ne

    origin = (0, reg - fpad)
    cover_end = (ne, reg - fpad + ne)

    idx = np.zeros((2 * H, nchunk, _CHUNK), np.int32)
    vzero = ([], [])
    for part, ents in enumerate((ent_k, ent_v)):
        full = ents + [None] * pad_total
        for j in range(nchunk):
            for r in range(_CHUNK):
                e = full[j * _CHUNK + r]
                if e is None and j * _CHUNK + r < ne:
                    vzero[part].append((j, r))
        base_idx = np.asarray([0 if e is None else e for e in full], np.int64)
        for h in range(H):
            idx[part * H + h] = (base_idx + h * T).astype(np.int32).reshape(
                nchunk, _CHUNK)

    zruns = ([], [])
    zend = (reg - fpad, 2 * reg)
    for part in range(2):
        z = cover_end[part]
        while z < zend[part]:
            m = min(_ZROWS, zend[part] - z)
            zruns[part].append((z, m))
            z += m
        assert cover_end[part] % 8 == 0

    return dict(idx=idx, nchunk=nchunk, last_m=last_m, origin=origin,
                vzero=vzero, zruns=zruns)


@functools.lru_cache(maxsize=None)
def _build_kernel(T, H, D):
    plan = _gather_plan(T, H)
    reg = _NSINK + _S
    outt = 2 * reg
    nunits = 2 * H
    assert nunits == 32, "one unit per TEC vector subcore"
    nchunk = plan["nchunk"]
    last_m = plan["last_m"]

    mesh = plsc.VectorSubcoreMesh(core_axis_name="c", subcore_axis_name="s")

    @functools.partial(
        pl.kernel,
        out_type=jax.ShapeDtypeStruct((H * outt, D), jnp.float32),
        mesh=mesh,
        scratch_types=(
            [pltpu.VMEM((nchunk, _CHUNK), jnp.int32)]
            + [pltpu.VMEM((_CHUNK, D), jnp.float32)] * _NBUF
            + [pltpu.VMEM((_ZROWS, D), jnp.float32)]
            + [pltpu.SemaphoreType.DMA] * (2 * _NBUF + 1)
        ),
    )
    def cache_fill(key_hbm, val_hbm, idx_hbm, zeros_hbm, out_hbm,
                   idx_v, *scratch):
        rows = scratch[:_NBUF]
        zbuf = scratch[_NBUF]
        gsems = scratch[_NBUF + 1:2 * _NBUF + 1]
        ssems = scratch[2 * _NBUF + 1:3 * _NBUF + 1]
        zsem = scratch[3 * _NBUF + 1]
        c = lax.axis_index("c")
        s = lax.axis_index("s")
        w = s * 2 + c
        part = w // H
        head = w % H
        hbase = head * outt

        pltpu.sync_copy(zeros_hbm, zbuf)
        pltpu.sync_copy(idx_hbm.at[w], idx_v)

        zero16 = jnp.zeros((16,), jnp.float32)

        def run(table, part_i):
            origin = plan["origin"][part_i]
            vzero = plan["vzero"][part_i]
            zdescs = [
                pltpu.async_copy(zbuf.at[pl.ds(0, m)],
                                 out_hbm.at[pl.ds(hbase + z, m)], zsem)
                for (z, m) in plan["zruns"][part_i]
            ]

            def fire_gather(j):
                return pltpu.async_copy(table.at[idx_v.at[j]],
                                        rows[j % _NBUF], gsems[j % _NBUF])

            def fire_store(j):
                m = _CHUNK if j < nchunk - 1 else last_m
                return pltpu.async_copy(
                    rows[j % _NBUF].at[pl.ds(0, m)],
                    out_hbm.at[pl.ds(hbase + origin + j * _CHUNK, m)],
                    ssems[j % _NBUF])

            gd = [None] * nchunk
            sd = [None] * nchunk
            for j in range(min(_NBUF - 1, nchunk)):
                gd[j] = fire_gather(j)
            for j in range(nchunk):
                gd[j].wait()
                for (jj, r) in vzero:
                    if jj == j:
                        for k in range(D // 16):
                            rows[j % _NBUF][r, pl.ds(k * 16, 16)] = zero16
                sd[j] = fire_store(j)
                nxt = j + _NBUF - 1
                if nxt < nchunk and gd[nxt] is None:
                    prev = nxt - _NBUF
                    if prev >= 0:
                        sd[prev].wait()
                        sd[prev] = None
                    gd[nxt] = fire_gather(nxt)
            for j in range(nchunk):
                if sd[j] is not None:
                    sd[j].wait()
            for d in zdescs:
                d.wait()

        @pl.when(part == 0)
        def _():
            run(key_hbm, 0)

        @pl.when(part == 1)
        def _():
            run(val_hbm, 1)

    return cache_fill


def kernel(key_states, value_states, layer_idx):
    del layer_idx
    B, H, T, D = key_states.shape
    assert B == 1
    plan = _gather_plan(T, H)
    fn = _build_kernel(T, H, D)
    out_flat = fn(
        key_states.reshape(H * T, D),
        value_states.reshape(H * T, D),
        jnp.asarray(plan["idx"]),
        jnp.zeros((_ZROWS, D), jnp.float32),
    )
    outt = 2 * (_NSINK + _S)
    return out_flat.reshape(B, H, outt, D)
